# baseline (device time: 182552 ns/iter reference)
import jax
import jax.numpy as jnp
from jax import lax
from jax.experimental import pallas as pl
from jax.experimental.pallas import tpu as pltpu

N_DEV = 16
B, SQ, D = 2, 256, 768
HQ_PER, DH = 8, 64
DLOC = HQ_PER * DH
ROWS = B * SQ
CHUNK = ROWS // N_DEV


def _modn(a):
    return lax.rem(a + 4 * N_DEV, N_DEV)


def kernel(x, Wq, Wo, K_ext, V_ext):
    me = lax.axis_index("i")
    h0 = me * HQ_PER
    K_loc = lax.dynamic_slice_in_dim(K_ext, h0, HQ_PER, axis=2)
    K_loc = K_loc.transpose(0, 2, 1, 3)
    V_loc = lax.dynamic_slice_in_dim(V_ext, h0, HQ_PER, axis=2)
    V_loc = V_loc.transpose(0, 2, 1, 3)

    def body(x_ref, wq_ref, wo_ref, k_ref, v_ref, out_ref,
             acc_ref, recv_ref, send_sems, recv_sems):
        me = lax.axis_index("i")
        left = _modn(me - 1)
        right = _modn(me + 1)

        barrier_sem = pltpu.get_barrier_semaphore()
        for nbr in (left, right):
            pl.semaphore_signal(
                barrier_sem, inc=1,
                device_id=(nbr,), device_id_type=pl.DeviceIdType.MESH,
            )
        pl.semaphore_wait(barrier_sem, 2)

        wq = wq_ref[...].astype(jnp.bfloat16)
        wo = wo_ref[...].astype(jnp.bfloat16)
        for b in range(B):
            xb = x_ref[b].astype(jnp.bfloat16)
            qb = jnp.dot(xb, wq, preferred_element_type=jnp.float32)
            o_cols = []
            for h in range(HQ_PER):
                q = (qb[:, h * DH:(h + 1) * DH] * 0.125).astype(jnp.bfloat16)
                k = k_ref[b, h].astype(jnp.bfloat16)
                v = v_ref[b, h].astype(jnp.bfloat16)
                s = jnp.dot(q, k.T, preferred_element_type=jnp.float32)
                m = jnp.max(s, axis=1, keepdims=True)
                p = jnp.exp(s - m)
                l = jnp.sum(p, axis=1, keepdims=True)
                o = jnp.dot(p.astype(jnp.bfloat16), v,
                            preferred_element_type=jnp.float32)
                o_cols.append(o / l)
            ob = jnp.concatenate(o_cols, axis=1)
            part = jnp.dot(ob.astype(jnp.bfloat16), wo,
                           preferred_element_type=jnp.float32)
            acc_ref[pl.ds(b * SQ, SQ), :] = part

        for t in range(2 * (N_DEV - 1)):
            slot = t % 2
            if t < N_DEV - 1:
                s = t
                send_c = _modn(me - s)
                recv_c = _modn(me - s - 1)
            else:
                s = t - (N_DEV - 1)
                send_c = _modn(me + 1 - s)
                recv_c = _modn(me - s)
            rdma = pltpu.make_async_remote_copy(
                src_ref=acc_ref.at[pl.ds(send_c * CHUNK, CHUNK), :],
                dst_ref=recv_ref.at[slot],
                send_sem=send_sems.at[slot],
                recv_sem=recv_sems.at[slot],
                device_id=(right,),
                device_id_type=pl.DeviceIdType.MESH,
            )
            rdma.start()
            rdma.wait()
            if t < N_DEV - 1:
                acc_ref[pl.ds(recv_c * CHUNK, CHUNK), :] = (
                    acc_ref[pl.ds(recv_c * CHUNK, CHUNK), :] + recv_ref[slot]
                )
            else:
                acc_ref[pl.ds(recv_c * CHUNK, CHUNK), :] = recv_ref[slot]

        for b in range(B):
            out_ref[b] = acc_ref[pl.ds(b * SQ, SQ), :]

    return pl.pallas_call(
        body,
        out_shape=jax.ShapeDtypeStruct((B, SQ, D), jnp.float32),
        in_specs=[pl.BlockSpec(memory_space=pltpu.VMEM)] * 5,
        out_specs=pl.BlockSpec(memory_space=pltpu.VMEM),
        scratch_shapes=[
            pltpu.VMEM((ROWS, D), jnp.float32),
            pltpu.VMEM((2, CHUNK, D), jnp.float32),
            pltpu.SemaphoreType.DMA((2,)),
            pltpu.SemaphoreType.DMA((2,)),
        ],
        compiler_params=pltpu.CompilerParams(collective_id=0),
    )(x, Wq, Wo, K_loc, V_loc)
